# baseline (speedup 1.0000x reference)
import functools

import jax
import jax.numpy as jnp
from jax import lax
from jax.experimental import pallas as pl
from jax.experimental.pallas import tpu as pltpu
from jax.experimental.pallas import tpu_sc as plsc

FL = 256
NL = 4096
ND = 2048
OC = 256
VIEWS = 6
CF = VIEWS * FL

SEG_W = 1 << 19
GARB = 4096
NTILE = 16
KD = 8


def _make_count_scatter(n, E):
    nseg = (n * n) // SEG_W
    segs_per_core = nseg // 2
    ET = E // NTILE
    NCH = ET // 128
    rounds = NCH // KD
    zw = SEG_W // NTILE

    mesh = plsc.VectorSubcoreMesh(core_axis_name="c", subcore_axis_name="s")

    @functools.partial(
        pl.kernel,
        out_type=jax.ShapeDtypeStruct((n * n,), jnp.float32),
        mesh=mesh,
        scratch_types=[
            pltpu.VMEM((ET,), jnp.int32),
            pltpu.VMEM((ET,), jnp.int32),
            pltpu.VMEM((NCH, 128), jnp.int32),
            pltpu.VMEM((KD, 128), jnp.int32),
            pltpu.VMEM((128,), jnp.float32),
            pltpu.VMEM_SHARED((SEG_W + GARB,), jnp.float32),
            pltpu.SemaphoreType.DMA,
        ],
    )
    def count_kernel(edges_hbm, ones_hbm, zeros_hbm, c_hbm, rows_v, cols_v,
                     flat_v, sidx_v, ones_v, seg_scr, sem):
        cid = lax.axis_index("c")
        sid = lax.axis_index("s")

        pltpu.sync_copy(edges_hbm.at[0, pl.ds(sid * ET, ET)], rows_v)
        pltpu.sync_copy(edges_hbm.at[1, pl.ds(sid * ET, ET)], cols_v)
        pltpu.sync_copy(ones_hbm, ones_v)

        def compute_flat(j, _):
            for g in range(8):
                r16 = rows_v[pl.ds(j * 128 + g * 16, 16)]
                c16 = cols_v[pl.ds(j * 128 + g * 16, 16)]
                flat_v[j, pl.ds(g * 16, 16)] = r16 * n + c16
            return 0

        lax.fori_loop(0, NCH, compute_flat, 0)

        def one_segment(si, _):
            seg_lo = (si * 2 + cid) * SEG_W
            pltpu.sync_copy(zeros_hbm, seg_scr.at[pl.ds(sid * zw, zw)])
            plsc.subcore_barrier()

            def one_round(rnd, _):
                descs = []
                for jb in range(KD):
                    j = rnd * KD + jb
                    for g in range(8):
                        f16 = flat_v[j, pl.ds(g * 16, 16)]
                        lidx = f16 - seg_lo
                        ok = (lidx >= 0) & (lidx < SEG_W)
                        lidx = jnp.where(ok, lidx, SEG_W + (f16 & (GARB - 1)))
                        sidx_v[jb, pl.ds(g * 16, 16)] = lidx
                    descs.append(pltpu.async_copy(
                        ones_v, seg_scr.at[sidx_v.at[jb]], sem, add=True))
                for d in descs:
                    d.wait()
                return 0

            lax.fori_loop(0, rounds, one_round, 0)
            plsc.subcore_barrier()
            pltpu.sync_copy(seg_scr.at[pl.ds(sid * zw, zw)],
                            c_hbm.at[pl.ds(seg_lo + sid * zw, zw)])
            plsc.subcore_barrier()
            return 0

        lax.fori_loop(0, segs_per_core, one_segment, 0)

    return count_kernel


def _u_body(x_ref, w_ref, dinv_ref, o_ref, ob_ref):
    u = dinv_ref[...] * jnp.dot(
        x_ref[...], w_ref[...], preferred_element_type=jnp.float32)
    o_ref[...] = u
    ob_ref[...] = u.astype(jnp.bfloat16)


def _compute_u(x, W, dinv_col):
    n = x.shape[0]
    bm = 1024
    return pl.pallas_call(
        _u_body,
        grid=(n // bm,),
        in_specs=[
            pl.BlockSpec((bm, FL), lambda i: (i, 0)),
            pl.BlockSpec((FL, FL), lambda i: (0, 0)),
            pl.BlockSpec((bm, 1), lambda i: (i, 0)),
        ],
        out_specs=[
            pl.BlockSpec((bm, FL), lambda i: (i, 0)),
            pl.BlockSpec((bm, FL), lambda i: (i, 0)),
        ],
        out_shape=[
            jax.ShapeDtypeStruct((n, FL), jnp.float32),
            jax.ShapeDtypeStruct((n, FL), jnp.bfloat16),
        ],
    )(x, W, dinv_col)


def _conv_body(s_ref, uk_ref, ur_ref, dinv_ref, b_ref, o_ref):
    k = pl.program_id(1)

    @pl.when(k == 0)
    def _init():
        o_ref[...] = jnp.zeros_like(o_ref)

    o_ref[...] += jax.lax.dot_general(
        s_ref[...], uk_ref[...], (((0,), (0,)), ((), ())),
        preferred_element_type=jnp.float32)

    @pl.when(k == pl.num_programs(1) - 1)
    def _epilogue():
        o_ref[...] = jax.nn.relu(
            dinv_ref[...] * (o_ref[...] + ur_ref[...]) + b_ref[...])


def _conv_apply(S, u_bf, u_f32, dinv_col, b_row, bn, bk):
    n = S.shape[0]
    return pl.pallas_call(
        _conv_body,
        grid=(n // bn, n // bk),
        in_specs=[
            pl.BlockSpec((bk, bn), lambda i, k: (k, i)),
            pl.BlockSpec((bk, FL), lambda i, k: (k, 0)),
            pl.BlockSpec((bn, FL), lambda i, k: (i, 0)),
            pl.BlockSpec((bn, 1), lambda i, k: (i, 0)),
            pl.BlockSpec((1, FL), lambda i, k: (0, 0)),
        ],
        out_specs=pl.BlockSpec((bn, FL), lambda i, k: (i, 0)),
        out_shape=jax.ShapeDtypeStruct((n, FL), jnp.float32),
    )(S, u_bf, u_f32, dinv_col, b_row)


def _rowsum_body(c_ref, o_ref):
    i = pl.program_id(0)

    @pl.when(i == 0)
    def _init():
        o_ref[...] = jnp.zeros_like(o_ref)

    o_ref[...] += jnp.sum(c_ref[...], axis=0, keepdims=True)


def _rowsum(concat):
    n = concat.shape[0]
    bm = 1024
    return pl.pallas_call(
        _rowsum_body,
        grid=(n // bm,),
        in_specs=[pl.BlockSpec((bm, CF), lambda i: (i, 0))],
        out_specs=pl.BlockSpec((1, CF), lambda i: (0, 0)),
        out_shape=jax.ShapeDtypeStruct((1, CF), jnp.float32),
    )(concat)


def _proj_body(c_ref, att_ref, w_ref, b_ref, o_ref):
    scaled = jax.nn.relu(att_ref[...] * c_ref[...])
    o_ref[...] = jax.lax.dot_general(
        scaled, w_ref[...], (((1,), (1,)), ((), ())),
        preferred_element_type=jnp.float32) + b_ref[...]


def _proj(concat, att_row, Wflat, b_row):
    n = concat.shape[0]
    bm = 1024
    return pl.pallas_call(
        _proj_body,
        grid=(n // bm,),
        in_specs=[
            pl.BlockSpec((bm, CF), lambda i: (i, 0)),
            pl.BlockSpec((1, CF), lambda i: (0, 0)),
            pl.BlockSpec((OC, CF), lambda i: (0, 0)),
            pl.BlockSpec((1, OC), lambda i: (0, 0)),
        ],
        out_specs=pl.BlockSpec((bm, OC), lambda i: (i, 0)),
        out_shape=jax.ShapeDtypeStruct((n, OC), jnp.float32),
    )(concat, att_row, Wflat, b_row)


def _final_body(x_ref, y_ref, o_ref):
    o_ref[...] = jax.lax.dot_general(
        x_ref[...], y_ref[...], (((1,), (1,)), ((), ())),
        preferred_element_type=jnp.float32)


def _final_matmul(xf, yf):
    bm, bn = 1024, 1024
    return pl.pallas_call(
        _final_body,
        grid=(NL // bm, ND // bn),
        in_specs=[
            pl.BlockSpec((bm, OC), lambda i, j: (i, 0)),
            pl.BlockSpec((bn, OC), lambda i, j: (j, 0)),
        ],
        out_specs=pl.BlockSpec((bm, bn), lambda i, j: (i, j)),
        out_shape=jax.ShapeDtypeStruct((NL, ND), jnp.float32),
    )(xf, yf)


def _sdeg_body(cnt_ref, mat_ref, s_ref, dinv_ref):
    s = cnt_ref[...].astype(jnp.float32) * mat_ref[...]
    s_ref[...] = s.astype(jnp.bfloat16)
    dinv_ref[...] = jax.lax.rsqrt(jnp.sum(s, axis=0, keepdims=True) + 1.0)


def _build_s_dinv(cnt, mat):
    n = mat.shape[0]
    bn = 256
    return pl.pallas_call(
        _sdeg_body,
        grid=(n // bn,),
        in_specs=[
            pl.BlockSpec((n, bn), lambda i: (0, i)),
            pl.BlockSpec((n, bn), lambda i: (0, i)),
        ],
        out_specs=[
            pl.BlockSpec((n, bn), lambda i: (0, i)),
            pl.BlockSpec((1, bn), lambda i: (0, i)),
        ],
        out_shape=[
            jax.ShapeDtypeStruct((n, n), jnp.bfloat16),
            jax.ShapeDtypeStruct((1, n), jnp.float32),
        ],
    )(cnt, mat)


def _gcn_branch(x, cnt, mat, W1, b1, W2, b2, n, bn, bk):
    S, dinv_row = _build_s_dinv(cnt, mat)
    dinv_col = dinv_row.reshape(n, 1)

    u1, u1b = _compute_u(x, W1, dinv_col)
    z1 = _conv_apply(S, u1b, u1, dinv_col, b1[None, :], bn, bk)
    u2, u2b = _compute_u(z1, W2, dinv_col)
    z2 = _conv_apply(S, u2b, u2, dinv_col, b2[None, :], bn, bk)
    return z1, z2


def _attention_mlp(rowsum, n, fc1_W, fc1_b, fc2_W, fc2_b):
    att = rowsum.reshape(VIEWS, FL).sum(axis=1) / (n * FL)
    att = jax.nn.relu(att @ fc1_W + fc1_b)
    att = jax.nn.sigmoid(att @ fc2_W + fc2_b)
    return jnp.repeat(att, FL)[None, :]


def kernel(x_l, x_d, lfs_edges, lfs_mat, lgs_edges, lgs_mat, lcs_edges, lcs_mat, dss_edges, dss_mat, dgs_edges, dgs_mat, dcs_edges, dcs_mat, params):
    p = params
    scat_l = _make_count_scatter(NL, lfs_edges.shape[1])
    scat_d = _make_count_scatter(ND, dss_edges.shape[1])
    ones_c = jnp.ones((128,), jnp.float32)
    zeros_c = jnp.zeros((SEG_W // NTILE,), jnp.float32)
    cnts = {
        "lfs": scat_l(lfs_edges, ones_c, zeros_c).reshape(NL, NL),
        "lgs": scat_l(lgs_edges, ones_c, zeros_c).reshape(NL, NL),
        "lcs": scat_l(lcs_edges, ones_c, zeros_c).reshape(NL, NL),
        "dss": scat_d(dss_edges, ones_c, zeros_c).reshape(ND, ND),
        "dgs": scat_d(dgs_edges, ones_c, zeros_c).reshape(ND, ND),
        "dcs": scat_d(dcs_edges, ones_c, zeros_c).reshape(ND, ND),
    }
    outs_l = []
    for nm, mat in [("lfs", lfs_mat), ("lgs", lgs_mat), ("lcs", lcs_mat)]:
        z1, z2 = _gcn_branch(x_l, cnts[nm], mat,
                             p[f"gcn_x1_{nm}_W"], p[f"gcn_x1_{nm}_b"],
                             p[f"gcn_x2_{nm}_W"], p[f"gcn_x2_{nm}_b"],
                             NL, 1024, 2048)
        outs_l += [z1, z2]
    outs_d = []
    for nm, mat in [("dss", dss_mat), ("dgs", dgs_mat), ("dcs", dcs_mat)]:
        z1, z2 = _gcn_branch(x_d, cnts[nm], mat,
                             p[f"gcn_y1_{nm}_W"], p[f"gcn_y1_{nm}_b"],
                             p[f"gcn_y2_{nm}_W"], p[f"gcn_y2_{nm}_b"],
                             ND, 1024, 2048)
        outs_d += [z1, z2]

    concat_x = jnp.concatenate(outs_l, axis=1)
    concat_y = jnp.concatenate(outs_d, axis=1)

    attx = _attention_mlp(_rowsum(concat_x), NL,
                          p["fc1_x_W"], p["fc1_x_b"], p["fc2_x_W"], p["fc2_x_b"])
    atty = _attention_mlp(_rowsum(concat_y), ND,
                          p["fc1_y_W"], p["fc1_y_b"], p["fc2_y_W"], p["fc2_y_b"])

    xf = _proj(concat_x, attx, p["cnn_x_W"].reshape(OC, CF), p["cnn_x_b"][None, :])
    yf = _proj(concat_y, atty, p["cnn_y_W"].reshape(OC, CF), p["cnn_y_b"][None, :])
    return _final_matmul(xf, yf)


# speedup vs baseline: 1.2744x; 1.2744x over previous
import functools

import jax
import jax.numpy as jnp
from jax import lax
from jax.experimental import pallas as pl
from jax.experimental.pallas import tpu as pltpu
from jax.experimental.pallas import tpu_sc as plsc

FL = 256
NL = 4096
ND = 2048
OC = 256
VIEWS = 6
CF = VIEWS * FL

SEG_W = 1 << 19
GARB = 4096
NTILE = 16
KD = 8


def _make_count_scatter(n, E):
    nseg = (n * n) // SEG_W
    segs_per_core = nseg // 2
    ET = E // NTILE
    NCH = ET // 128
    rounds = NCH // KD
    zw = SEG_W // NTILE

    mesh = plsc.VectorSubcoreMesh(core_axis_name="c", subcore_axis_name="s")

    @functools.partial(
        pl.kernel,
        out_type=jax.ShapeDtypeStruct((n * n,), jnp.float32),
        mesh=mesh,
        scratch_types=[
            pltpu.VMEM((ET,), jnp.int32),
            pltpu.VMEM((ET,), jnp.int32),
            pltpu.VMEM((NCH, 128), jnp.int32),
            pltpu.VMEM((KD, 128), jnp.int32),
            pltpu.VMEM((128,), jnp.float32),
            pltpu.VMEM((zw,), jnp.float32),
            pltpu.VMEM_SHARED((SEG_W + GARB,), jnp.float32),
            pltpu.SemaphoreType.DMA,
        ],
    )
    def count_kernel(edges_hbm, ones_hbm, zeros_hbm, c_hbm, rows_v, cols_v,
                     flat_v, sidx_v, ones_v, zero_v, seg_scr, sem):
        cid = lax.axis_index("c")
        sid = lax.axis_index("s")

        pltpu.sync_copy(edges_hbm.at[0, pl.ds(sid * ET, ET)], rows_v)
        pltpu.sync_copy(edges_hbm.at[1, pl.ds(sid * ET, ET)], cols_v)
        pltpu.sync_copy(ones_hbm, ones_v)

        def compute_flat(j, _):
            for g in range(8):
                r16 = rows_v[pl.ds(j * 128 + g * 16, 16)]
                c16 = cols_v[pl.ds(j * 128 + g * 16, 16)]
                flat_v[j, pl.ds(g * 16, 16)] = r16 * n + c16
            return 0

        lax.fori_loop(0, NCH, compute_flat, 0)
        pltpu.sync_copy(zeros_hbm, zero_v)

        def one_segment(si, _):
            seg_lo = (si * 2 + cid) * SEG_W
            pltpu.sync_copy(zero_v, seg_scr.at[pl.ds(sid * zw, zw)])
            plsc.subcore_barrier()

            def one_round(rnd, _):
                descs = []
                for jb in range(KD):
                    j = rnd * KD + jb
                    for g in range(8):
                        f16 = flat_v[j, pl.ds(g * 16, 16)]
                        lidx = f16 - seg_lo
                        ok = (lidx >= 0) & (lidx < SEG_W)
                        lidx = jnp.where(ok, lidx, SEG_W + (f16 & (GARB - 1)))
                        sidx_v[jb, pl.ds(g * 16, 16)] = lidx
                    descs.append(pltpu.async_copy(
                        ones_v, seg_scr.at[sidx_v.at[jb]], sem, add=True))
                for d in descs:
                    d.wait()
                return 0

            lax.fori_loop(0, rounds, one_round, 0)
            plsc.subcore_barrier()
            pltpu.sync_copy(seg_scr.at[pl.ds(sid * zw, zw)],
                            c_hbm.at[pl.ds(seg_lo + sid * zw, zw)])
            plsc.subcore_barrier()
            return 0

        lax.fori_loop(0, segs_per_core, one_segment, 0)

    return count_kernel


def _u_body(x_ref, w_ref, dinv_ref, o_ref, ob_ref):
    u = dinv_ref[...] * jnp.dot(
        x_ref[...], w_ref[...], preferred_element_type=jnp.float32)
    o_ref[...] = u
    ob_ref[...] = u.astype(jnp.bfloat16)


def _compute_u(x, W, dinv_col):
    n = x.shape[0]
    bm = 1024
    return pl.pallas_call(
        _u_body,
        grid=(n // bm,),
        in_specs=[
            pl.BlockSpec((bm, FL), lambda i: (i, 0)),
            pl.BlockSpec((FL, FL), lambda i: (0, 0)),
            pl.BlockSpec((bm, 1), lambda i: (i, 0)),
        ],
        out_specs=[
            pl.BlockSpec((bm, FL), lambda i: (i, 0)),
            pl.BlockSpec((bm, FL), lambda i: (i, 0)),
        ],
        out_shape=[
            jax.ShapeDtypeStruct((n, FL), jnp.float32),
            jax.ShapeDtypeStruct((n, FL), jnp.bfloat16),
        ],
    )(x, W, dinv_col)


def _conv_body(s_ref, uk_ref, ur_ref, dinv_ref, b_ref, o_ref):
    k = pl.program_id(1)

    @pl.when(k == 0)
    def _init():
        o_ref[...] = jnp.zeros_like(o_ref)

    o_ref[...] += jax.lax.dot_general(
        s_ref[...], uk_ref[...], (((0,), (0,)), ((), ())),
        preferred_element_type=jnp.float32)

    @pl.when(k == pl.num_programs(1) - 1)
    def _epilogue():
        o_ref[...] = jax.nn.relu(
            dinv_ref[...] * (o_ref[...] + ur_ref[...]) + b_ref[...])


def _conv_apply(S, u_bf, u_f32, dinv_col, b_row, bn, bk):
    n = S.shape[0]
    return pl.pallas_call(
        _conv_body,
        grid=(n // bn, n // bk),
        in_specs=[
            pl.BlockSpec((bk, bn), lambda i, k: (k, i)),
            pl.BlockSpec((bk, FL), lambda i, k: (k, 0)),
            pl.BlockSpec((bn, FL), lambda i, k: (i, 0)),
            pl.BlockSpec((bn, 1), lambda i, k: (i, 0)),
            pl.BlockSpec((1, FL), lambda i, k: (0, 0)),
        ],
        out_specs=pl.BlockSpec((bn, FL), lambda i, k: (i, 0)),
        out_shape=jax.ShapeDtypeStruct((n, FL), jnp.float32),
    )(S, u_bf, u_f32, dinv_col, b_row)


def _rowsum_body(c_ref, o_ref):
    i = pl.program_id(0)

    @pl.when(i == 0)
    def _init():
        o_ref[...] = jnp.zeros_like(o_ref)

    o_ref[...] += jnp.sum(c_ref[...], axis=0, keepdims=True)


def _rowsum(concat):
    n = concat.shape[0]
    bm = 1024
    return pl.pallas_call(
        _rowsum_body,
        grid=(n // bm,),
        in_specs=[pl.BlockSpec((bm, CF), lambda i: (i, 0))],
        out_specs=pl.BlockSpec((1, CF), lambda i: (0, 0)),
        out_shape=jax.ShapeDtypeStruct((1, CF), jnp.float32),
    )(concat)


def _proj_body(c_ref, att_ref, w_ref, b_ref, o_ref):
    scaled = jax.nn.relu(att_ref[...] * c_ref[...])
    o_ref[...] = jax.lax.dot_general(
        scaled, w_ref[...], (((1,), (1,)), ((), ())),
        preferred_element_type=jnp.float32) + b_ref[...]


def _proj(concat, att_row, Wflat, b_row):
    n = concat.shape[0]
    bm = 1024
    return pl.pallas_call(
        _proj_body,
        grid=(n // bm,),
        in_specs=[
            pl.BlockSpec((bm, CF), lambda i: (i, 0)),
            pl.BlockSpec((1, CF), lambda i: (0, 0)),
            pl.BlockSpec((OC, CF), lambda i: (0, 0)),
            pl.BlockSpec((1, OC), lambda i: (0, 0)),
        ],
        out_specs=pl.BlockSpec((bm, OC), lambda i: (i, 0)),
        out_shape=jax.ShapeDtypeStruct((n, OC), jnp.float32),
    )(concat, att_row, Wflat, b_row)


def _final_body(x_ref, y_ref, o_ref):
    o_ref[...] = jax.lax.dot_general(
        x_ref[...], y_ref[...], (((1,), (1,)), ((), ())),
        preferred_element_type=jnp.float32)


def _final_matmul(xf, yf):
    bm, bn = 1024, 1024
    return pl.pallas_call(
        _final_body,
        grid=(NL // bm, ND // bn),
        in_specs=[
            pl.BlockSpec((bm, OC), lambda i, j: (i, 0)),
            pl.BlockSpec((bn, OC), lambda i, j: (j, 0)),
        ],
        out_specs=pl.BlockSpec((bm, bn), lambda i, j: (i, j)),
        out_shape=jax.ShapeDtypeStruct((NL, ND), jnp.float32),
    )(xf, yf)


def _sdeg_body(cnt_ref, mat_ref, s_ref, dinv_ref):
    s = cnt_ref[...].astype(jnp.float32) * mat_ref[...]
    s_ref[...] = s.astype(jnp.bfloat16)
    dinv_ref[...] = jax.lax.rsqrt(jnp.sum(s, axis=0, keepdims=True) + 1.0)


def _build_s_dinv(cnt, mat):
    n = mat.shape[0]
    bn = 256
    return pl.pallas_call(
        _sdeg_body,
        grid=(n // bn,),
        in_specs=[
            pl.BlockSpec((n, bn), lambda i: (0, i)),
            pl.BlockSpec((n, bn), lambda i: (0, i)),
        ],
        out_specs=[
            pl.BlockSpec((n, bn), lambda i: (0, i)),
            pl.BlockSpec((1, bn), lambda i: (0, i)),
        ],
        out_shape=[
            jax.ShapeDtypeStruct((n, n), jnp.bfloat16),
            jax.ShapeDtypeStruct((1, n), jnp.float32),
        ],
    )(cnt, mat)


def _gcn_branch(x, cnt, mat, W1, b1, W2, b2, n, bn, bk):
    S, dinv_row = _build_s_dinv(cnt, mat)
    dinv_col = dinv_row.reshape(n, 1)

    u1, u1b = _compute_u(x, W1, dinv_col)
    z1 = _conv_apply(S, u1b, u1, dinv_col, b1[None, :], bn, bk)
    u2, u2b = _compute_u(z1, W2, dinv_col)
    z2 = _conv_apply(S, u2b, u2, dinv_col, b2[None, :], bn, bk)
    return z1, z2


def _attention_mlp(rowsum, n, fc1_W, fc1_b, fc2_W, fc2_b):
    att = rowsum.reshape(VIEWS, FL).sum(axis=1) / (n * FL)
    att = jax.nn.relu(att @ fc1_W + fc1_b)
    att = jax.nn.sigmoid(att @ fc2_W + fc2_b)
    return jnp.repeat(att, FL)[None, :]


def kernel(x_l, x_d, lfs_edges, lfs_mat, lgs_edges, lgs_mat, lcs_edges, lcs_mat, dss_edges, dss_mat, dgs_edges, dgs_mat, dcs_edges, dcs_mat, params):
    p = params
    scat_l = _make_count_scatter(NL, lfs_edges.shape[1])
    scat_d = _make_count_scatter(ND, dss_edges.shape[1])
    ones_c = jnp.ones((128,), jnp.float32)
    zeros_c = jnp.zeros((SEG_W // NTILE,), jnp.float32)
    cnts = {
        "lfs": scat_l(lfs_edges, ones_c, zeros_c).reshape(NL, NL),
        "lgs": scat_l(lgs_edges, ones_c, zeros_c).reshape(NL, NL),
        "lcs": scat_l(lcs_edges, ones_c, zeros_c).reshape(NL, NL),
        "dss": scat_d(dss_edges, ones_c, zeros_c).reshape(ND, ND),
        "dgs": scat_d(dgs_edges, ones_c, zeros_c).reshape(ND, ND),
        "dcs": scat_d(dcs_edges, ones_c, zeros_c).reshape(ND, ND),
    }
    outs_l = []
    for nm, mat in [("lfs", lfs_mat), ("lgs", lgs_mat), ("lcs", lcs_mat)]:
        z1, z2 = _gcn_branch(x_l, cnts[nm], mat,
                             p[f"gcn_x1_{nm}_W"], p[f"gcn_x1_{nm}_b"],
                             p[f"gcn_x2_{nm}_W"], p[f"gcn_x2_{nm}_b"],
                             NL, 1024, 2048)
        outs_l += [z1, z2]
    outs_d = []
    for nm, mat in [("dss", dss_mat), ("dgs", dgs_mat), ("dcs", dcs_mat)]:
        z1, z2 = _gcn_branch(x_d, cnts[nm], mat,
                             p[f"gcn_y1_{nm}_W"], p[f"gcn_y1_{nm}_b"],
                             p[f"gcn_y2_{nm}_W"], p[f"gcn_y2_{nm}_b"],
                             ND, 1024, 2048)
        outs_d += [z1, z2]

    concat_x = jnp.concatenate(outs_l, axis=1)
    concat_y = jnp.concatenate(outs_d, axis=1)

    attx = _attention_mlp(_rowsum(concat_x), NL,
                          p["fc1_x_W"], p["fc1_x_b"], p["fc2_x_W"], p["fc2_x_b"])
    atty = _attention_mlp(_rowsum(concat_y), ND,
                          p["fc1_y_W"], p["fc1_y_b"], p["fc2_y_W"], p["fc2_y_b"])

    xf = _proj(concat_x, attx, p["cnn_x_W"].reshape(OC, CF), p["cnn_x_b"][None, :])
    yf = _proj(concat_y, atty, p["cnn_y_W"].reshape(OC, CF), p["cnn_y_b"][None, :])
    return _final_matmul(xf, yf)


# speedup vs baseline: 1.2751x; 1.0006x over previous
import functools

import jax
import jax.numpy as jnp
from jax import lax
from jax.experimental import pallas as pl
from jax.experimental.pallas import tpu as pltpu
from jax.experimental.pallas import tpu_sc as plsc

FL = 256
NL = 4096
ND = 2048
OC = 256
VIEWS = 6
CF = VIEWS * FL

SEG_W = 1 << 19
GARB = 4096
NTILE = 16
KD = 16


def _make_count_scatter(n, E):
    nseg = (n * n) // SEG_W
    segs_per_core = nseg // 2
    ET = E // NTILE
    NCH = ET // 128
    rounds = NCH // KD
    zw = SEG_W // NTILE

    mesh = plsc.VectorSubcoreMesh(core_axis_name="c", subcore_axis_name="s")

    @functools.partial(
        pl.kernel,
        out_type=jax.ShapeDtypeStruct((n * n,), jnp.float32),
        mesh=mesh,
        scratch_types=[
            pltpu.VMEM((ET,), jnp.int32),
            pltpu.VMEM((ET,), jnp.int32),
            pltpu.VMEM((NCH, 128), jnp.int32),
            pltpu.VMEM((KD, 128), jnp.int32),
            pltpu.VMEM((128,), jnp.float32),
            pltpu.VMEM((zw,), jnp.float32),
            pltpu.VMEM_SHARED((SEG_W + GARB,), jnp.float32),
            pltpu.SemaphoreType.DMA,
        ],
    )
    def count_kernel(edges_hbm, ones_hbm, zeros_hbm, c_hbm, rows_v, cols_v,
                     flat_v, sidx_v, ones_v, zero_v, seg_scr, sem):
        cid = lax.axis_index("c")
        sid = lax.axis_index("s")

        pltpu.sync_copy(edges_hbm.at[0, pl.ds(sid * ET, ET)], rows_v)
        pltpu.sync_copy(edges_hbm.at[1, pl.ds(sid * ET, ET)], cols_v)
        pltpu.sync_copy(ones_hbm, ones_v)

        def compute_flat(j, _):
            for g in range(8):
                r16 = rows_v[pl.ds(j * 128 + g * 16, 16)]
                c16 = cols_v[pl.ds(j * 128 + g * 16, 16)]
                flat_v[j, pl.ds(g * 16, 16)] = r16 * n + c16
            return 0

        lax.fori_loop(0, NCH, compute_flat, 0)
        pltpu.sync_copy(zeros_hbm, zero_v)
        pltpu.sync_copy(zero_v, seg_scr.at[pl.ds(sid * zw, zw)])
        plsc.subcore_barrier()

        def one_segment(si, _):
            seg_lo = (si * 2 + cid) * SEG_W

            def one_round(rnd, _):
                descs = []
                for jb in range(KD):
                    j = rnd * KD + jb
                    for g in range(8):
                        f16 = flat_v[j, pl.ds(g * 16, 16)]
                        lidx = f16 - seg_lo
                        ok = (lidx >= 0) & (lidx < SEG_W)
                        lidx = jnp.where(ok, lidx, SEG_W + (f16 & (GARB - 1)))
                        sidx_v[jb, pl.ds(g * 16, 16)] = lidx
                    descs.append(pltpu.async_copy(
                        ones_v, seg_scr.at[sidx_v.at[jb]], sem, add=True))
                for d in descs:
                    d.wait()
                return 0

            lax.fori_loop(0, rounds, one_round, 0)
            plsc.subcore_barrier()
            pltpu.sync_copy(seg_scr.at[pl.ds(sid * zw, zw)],
                            c_hbm.at[pl.ds(seg_lo + sid * zw, zw)])
            pltpu.sync_copy(zero_v, seg_scr.at[pl.ds(sid * zw, zw)])
            plsc.subcore_barrier()
            return 0

        lax.fori_loop(0, segs_per_core, one_segment, 0)

    return count_kernel


def _u_body(x_ref, w_ref, dinv_ref, o_ref, ob_ref):
    u = dinv_ref[...] * jnp.dot(
        x_ref[...], w_ref[...], preferred_element_type=jnp.float32)
    o_ref[...] = u
    ob_ref[...] = u.astype(jnp.bfloat16)


def _compute_u(x, W, dinv_col):
    n = x.shape[0]
    bm = 1024
    return pl.pallas_call(
        _u_body,
        grid=(n // bm,),
        in_specs=[
            pl.BlockSpec((bm, FL), lambda i: (i, 0)),
            pl.BlockSpec((FL, FL), lambda i: (0, 0)),
            pl.BlockSpec((bm, 1), lambda i: (i, 0)),
        ],
        out_specs=[
            pl.BlockSpec((bm, FL), lambda i: (i, 0)),
            pl.BlockSpec((bm, FL), lambda i: (i, 0)),
        ],
        out_shape=[
            jax.ShapeDtypeStruct((n, FL), jnp.float32),
            jax.ShapeDtypeStruct((n, FL), jnp.bfloat16),
        ],
    )(x, W, dinv_col)


def _conv_body(s_ref, uk_ref, ur_ref, dinv_ref, b_ref, o_ref):
    k = pl.program_id(1)

    @pl.when(k == 0)
    def _init():
        o_ref[...] = jnp.zeros_like(o_ref)

    o_ref[...] += jax.lax.dot_general(
        s_ref[...], uk_ref[...], (((0,), (0,)), ((), ())),
        preferred_element_type=jnp.float32)

    @pl.when(k == pl.num_programs(1) - 1)
    def _epilogue():
        o_ref[...] = jax.nn.relu(
            dinv_ref[...] * (o_ref[...] + ur_ref[...]) + b_ref[...])


def _conv_apply(S, u_bf, u_f32, dinv_col, b_row, bn, bk):
    n = S.shape[0]
    return pl.pallas_call(
        _conv_body,
        grid=(n // bn, n // bk),
        in_specs=[
            pl.BlockSpec((bk, bn), lambda i, k: (k, i)),
            pl.BlockSpec((bk, FL), lambda i, k: (k, 0)),
            pl.BlockSpec((bn, FL), lambda i, k: (i, 0)),
            pl.BlockSpec((bn, 1), lambda i, k: (i, 0)),
            pl.BlockSpec((1, FL), lambda i, k: (0, 0)),
        ],
        out_specs=pl.BlockSpec((bn, FL), lambda i, k: (i, 0)),
        out_shape=jax.ShapeDtypeStruct((n, FL), jnp.float32),
    )(S, u_bf, u_f32, dinv_col, b_row)


def _rowsum_body(c_ref, o_ref):
    i = pl.program_id(0)

    @pl.when(i == 0)
    def _init():
        o_ref[...] = jnp.zeros_like(o_ref)

    o_ref[...] += jnp.sum(c_ref[...], axis=0, keepdims=True)


def _rowsum(concat):
    n = concat.shape[0]
    bm = 1024
    return pl.pallas_call(
        _rowsum_body,
        grid=(n // bm,),
        in_specs=[pl.BlockSpec((bm, CF), lambda i: (i, 0))],
        out_specs=pl.BlockSpec((1, CF), lambda i: (0, 0)),
        out_shape=jax.ShapeDtypeStruct((1, CF), jnp.float32),
    )(concat)


def _proj_body(c_ref, att_ref, w_ref, b_ref, o_ref):
    scaled = jax.nn.relu(att_ref[...] * c_ref[...])
    o_ref[...] = jax.lax.dot_general(
        scaled, w_ref[...], (((1,), (1,)), ((), ())),
        preferred_element_type=jnp.float32) + b_ref[...]


def _proj(concat, att_row, Wflat, b_row):
    n = concat.shape[0]
    bm = 1024
    return pl.pallas_call(
        _proj_body,
        grid=(n // bm,),
        in_specs=[
            pl.BlockSpec((bm, CF), lambda i: (i, 0)),
            pl.BlockSpec((1, CF), lambda i: (0, 0)),
            pl.BlockSpec((OC, CF), lambda i: (0, 0)),
            pl.BlockSpec((1, OC), lambda i: (0, 0)),
        ],
        out_specs=pl.BlockSpec((bm, OC), lambda i: (i, 0)),
        out_shape=jax.ShapeDtypeStruct((n, OC), jnp.float32),
    )(concat, att_row, Wflat, b_row)


def _final_body(x_ref, y_ref, o_ref):
    o_ref[...] = jax.lax.dot_general(
        x_ref[...], y_ref[...], (((1,), (1,)), ((), ())),
        preferred_element_type=jnp.float32)


def _final_matmul(xf, yf):
    bm, bn = 1024, 1024
    return pl.pallas_call(
        _final_body,
        grid=(NL // bm, ND // bn),
        in_specs=[
            pl.BlockSpec((bm, OC), lambda i, j: (i, 0)),
            pl.BlockSpec((bn, OC), lambda i, j: (j, 0)),
        ],
        out_specs=pl.BlockSpec((bm, bn), lambda i, j: (i, j)),
        out_shape=jax.ShapeDtypeStruct((NL, ND), jnp.float32),
    )(xf, yf)


def _sdeg_body(cnt_ref, mat_ref, s_ref, dinv_ref):
    s = cnt_ref[...].astype(jnp.float32) * mat_ref[...]
    s_ref[...] = s.astype(jnp.bfloat16)
    dinv_ref[...] = jax.lax.rsqrt(jnp.sum(s, axis=0, keepdims=True) + 1.0)


def _build_s_dinv(cnt, mat):
    n = mat.shape[0]
    bn = 256
    return pl.pallas_call(
        _sdeg_body,
        grid=(n // bn,),
        in_specs=[
            pl.BlockSpec((n, bn), lambda i: (0, i)),
            pl.BlockSpec((n, bn), lambda i: (0, i)),
        ],
        out_specs=[
            pl.BlockSpec((n, bn), lambda i: (0, i)),
            pl.BlockSpec((1, bn), lambda i: (0, i)),
        ],
        out_shape=[
            jax.ShapeDtypeStruct((n, n), jnp.bfloat16),
            jax.ShapeDtypeStruct((1, n), jnp.float32),
        ],
    )(cnt, mat)


def _gcn_branch(x, cnt, mat, W1, b1, W2, b2, n, bn, bk):
    S, dinv_row = _build_s_dinv(cnt, mat)
    dinv_col = dinv_row.reshape(n, 1)

    u1, u1b = _compute_u(x, W1, dinv_col)
    z1 = _conv_apply(S, u1b, u1, dinv_col, b1[None, :], bn, bk)
    u2, u2b = _compute_u(z1, W2, dinv_col)
    z2 = _conv_apply(S, u2b, u2, dinv_col, b2[None, :], bn, bk)
    return z1, z2


def _attention_mlp(rowsum, n, fc1_W, fc1_b, fc2_W, fc2_b):
    att = rowsum.reshape(VIEWS, FL).sum(axis=1) / (n * FL)
    att = jax.nn.relu(att @ fc1_W + fc1_b)
    att = jax.nn.sigmoid(att @ fc2_W + fc2_b)
    return jnp.repeat(att, FL)[None, :]


def kernel(x_l, x_d, lfs_edges, lfs_mat, lgs_edges, lgs_mat, lcs_edges, lcs_mat, dss_edges, dss_mat, dgs_edges, dgs_mat, dcs_edges, dcs_mat, params):
    p = params
    scat_l = _make_count_scatter(NL, lfs_edges.shape[1])
    scat_d = _make_count_scatter(ND, dss_edges.shape[1])
    ones_c = jnp.ones((128,), jnp.float32)
    zeros_c = jnp.zeros((SEG_W // NTILE,), jnp.float32)
    cnts = {
        "lfs": scat_l(lfs_edges, ones_c, zeros_c).reshape(NL, NL),
        "lgs": scat_l(lgs_edges, ones_c, zeros_c).reshape(NL, NL),
        "lcs": scat_l(lcs_edges, ones_c, zeros_c).reshape(NL, NL),
        "dss": scat_d(dss_edges, ones_c, zeros_c).reshape(ND, ND),
        "dgs": scat_d(dgs_edges, ones_c, zeros_c).reshape(ND, ND),
        "dcs": scat_d(dcs_edges, ones_c, zeros_c).reshape(ND, ND),
    }
    outs_l = []
    for nm, mat in [("lfs", lfs_mat), ("lgs", lgs_mat), ("lcs", lcs_mat)]:
        z1, z2 = _gcn_branch(x_l, cnts[nm], mat,
                             p[f"gcn_x1_{nm}_W"], p[f"gcn_x1_{nm}_b"],
                             p[f"gcn_x2_{nm}_W"], p[f"gcn_x2_{nm}_b"],
                             NL, 1024, 2048)
        outs_l += [z1, z2]
    outs_d = []
    for nm, mat in [("dss", dss_mat), ("dgs", dgs_mat), ("dcs", dcs_mat)]:
        z1, z2 = _gcn_branch(x_d, cnts[nm], mat,
                             p[f"gcn_y1_{nm}_W"], p[f"gcn_y1_{nm}_b"],
                             p[f"gcn_y2_{nm}_W"], p[f"gcn_y2_{nm}_b"],
                             ND, 1024, 2048)
        outs_d += [z1, z2]

    concat_x = jnp.concatenate(outs_l, axis=1)
    concat_y = jnp.concatenate(outs_d, axis=1)

    attx = _attention_mlp(_rowsum(concat_x), NL,
                          p["fc1_x_W"], p["fc1_x_b"], p["fc2_x_W"], p["fc2_x_b"])
    atty = _attention_mlp(_rowsum(concat_y), ND,
                          p["fc1_y_W"], p["fc1_y_b"], p["fc2_y_W"], p["fc2_y_b"])

    xf = _proj(concat_x, attx, p["cnn_x_W"].reshape(OC, CF), p["cnn_x_b"][None, :])
    yf = _proj(concat_y, atty, p["cnn_y_W"].reshape(OC, CF), p["cnn_y_b"][None, :])
    return _final_matmul(xf, yf)
